# baseline (speedup 1.0000x reference)
import functools

import jax
import jax.numpy as jnp
from jax import lax
from jax.experimental import pallas as pl
from jax.experimental.pallas import tpu as pltpu
from jax.experimental.pallas import tpu_sc as plsc

B, T, D = 1, 2048, 768
E, K, DF = 8, 2, 3072
N = B * T
BLK = 256
NBLK = 24
NP = BLK * NBLK
CH = 128
NW = 32
CHUNK = N // NW


def _router_body(x_ref, rw_ref, pos0_ref, pos1_ref, w0_ref, w1_ref, bem_ref,
                 pf_ref, sc_ref, sp_ref):
    xf = x_ref[...]
    logits = jnp.dot(xf, rw_ref[...], preferred_element_type=jnp.float32)
    ids = jax.lax.broadcasted_iota(jnp.int32, (N, E), 1)
    m0 = jnp.max(logits, axis=1, keepdims=True)
    e0 = jnp.min(jnp.where(logits == m0, ids, E), axis=1, keepdims=True)
    oh0 = ids == e0
    l2 = jnp.where(oh0, -jnp.inf, logits)
    m1 = jnp.max(l2, axis=1, keepdims=True)
    e1 = jnp.min(jnp.where(l2 == m1, ids, E), axis=1, keepdims=True)
    oh1 = ids == e1
    t = jnp.exp(m1 - m0)
    w_hi = 1.0 / (1.0 + t)
    w_lo = 1.0 - w_hi
    ones16 = jnp.ones((1, 16), jnp.float32)
    w0_ref[...] = w_hi * ones16
    w1_ref[...] = w_lo * ones16

    f0 = oh0.astype(jnp.float32)
    f1 = oh1.astype(jnp.float32)
    sub = jax.lax.broadcasted_iota(jnp.int32, (CH, CH), 0)
    lan = jax.lax.broadcasted_iota(jnp.int32, (CH, CH), 1)
    tri = (lan < sub).astype(jnp.float32)
    run0 = jnp.zeros((1, E), jnp.float32)
    run1 = jnp.zeros((1, E), jnp.float32)
    cum0 = []
    cum1 = []
    for c in range(N // CH):
        a0 = f0[c * CH:(c + 1) * CH, :]
        a1 = f1[c * CH:(c + 1) * CH, :]
        cum0.append(jnp.dot(tri, a0, preferred_element_type=jnp.float32) + run0)
        cum1.append(jnp.dot(tri, a1, preferred_element_type=jnp.float32) + run1)
        run0 = run0 + jnp.sum(a0, axis=0, keepdims=True)
        run1 = run1 + jnp.sum(a1, axis=0, keepdims=True)
    cum0 = jnp.concatenate(cum0, axis=0)
    cum1 = jnp.concatenate(cum1, axis=0)
    cnt0 = run0
    total = run0 + run1
    padded = jnp.ceil(total / BLK) * BLK
    ii = jax.lax.broadcasted_iota(jnp.int32, (E, E), 0)
    jj = jax.lax.broadcasted_iota(jnp.int32, (E, E), 1)
    triu = (ii < jj).astype(jnp.float32)
    off = jnp.dot(padded, triu, preferred_element_type=jnp.float32)

    base0 = jnp.sum(f0 * off, axis=1, keepdims=True)
    base1 = jnp.sum(f1 * (off + cnt0), axis=1, keepdims=True)
    rank0 = jnp.sum(f0 * cum0, axis=1, keepdims=True)
    rank1 = jnp.sum(f1 * cum1, axis=1, keepdims=True)
    pos0_ref[...] = (base0 + rank0).astype(jnp.int32)
    pos1_ref[...] = (base1 + rank1).astype(jnp.int32)

    bs = jax.lax.broadcasted_iota(jnp.int32, (NBLK, 1), 0).astype(
        jnp.float32) * BLK
    bem = jnp.sum((off <= bs).astype(jnp.int32), axis=1, keepdims=True) - 1
    tot_pad = jnp.sum(padded, axis=1, keepdims=True)
    bem = jnp.where(bs < tot_pad, bem, -1)
    bem_ref[...] = bem

    present = (total > 0.0).astype(jnp.float32)
    rank = jnp.dot(present, triu, preferred_element_type=jnp.float32)
    slot_row = jnp.remainder(rank.astype(jnp.int32), 2)
    jb = jax.lax.broadcasted_iota(jnp.int32, (NBLK, E), 1)
    cand = (jb > bem) & (present > 0.0)
    pf = jnp.min(jnp.where(cand, jb, E), axis=1, keepdims=True)
    pf = jnp.where(pf == E, -1, pf)
    ohb = (jb == bem).astype(jnp.int32)
    ohp = (jb == pf).astype(jnp.int32)
    pf_ref[...] = pf
    sc_ref[...] = jnp.sum(ohb * slot_row, axis=1, keepdims=True)
    sp_ref[...] = jnp.sum(ohp * slot_row, axis=1, keepdims=True)


def _router(xf, router_w):
    return pl.pallas_call(
        _router_body,
        in_specs=[
            pl.BlockSpec((N, D), lambda: (0, 0)),
            pl.BlockSpec((D, E), lambda: (0, 0)),
        ],
        out_specs=[
            pl.BlockSpec((N, 1), lambda: (0, 0)),
            pl.BlockSpec((N, 1), lambda: (0, 0)),
            pl.BlockSpec((N, 16), lambda: (0, 0)),
            pl.BlockSpec((N, 16), lambda: (0, 0)),
            pl.BlockSpec((NBLK, 1), lambda: (0, 0)),
            pl.BlockSpec((NBLK, 1), lambda: (0, 0)),
            pl.BlockSpec((NBLK, 1), lambda: (0, 0)),
            pl.BlockSpec((NBLK, 1), lambda: (0, 0)),
        ],
        out_shape=[
            jax.ShapeDtypeStruct((N, 1), jnp.int32),
            jax.ShapeDtypeStruct((N, 1), jnp.int32),
            jax.ShapeDtypeStruct((N, 16), jnp.float32),
            jax.ShapeDtypeStruct((N, 16), jnp.float32),
            jax.ShapeDtypeStruct((NBLK, 1), jnp.int32),
            jax.ShapeDtypeStruct((NBLK, 1), jnp.int32),
            jax.ShapeDtypeStruct((NBLK, 1), jnp.int32),
            jax.ShapeDtypeStruct((NBLK, 1), jnp.int32),
        ],
    )(xf, router_w)


@functools.cache
def _sc_kernels():
    mesh = plsc.VectorSubcoreMesh(core_axis_name="c", subcore_axis_name="s")

    @functools.partial(
        pl.kernel, mesh=mesh,
        out_type=jax.ShapeDtypeStruct((NP, D), jnp.float32),
        scratch_types=[
            pltpu.VMEM((CHUNK, D), jnp.float32),
            pltpu.VMEM((CHUNK,), jnp.int32),
            pltpu.VMEM((CHUNK,), jnp.int32),
            pltpu.SemaphoreType.DMA,
            pltpu.SemaphoreType.DMA,
        ],
    )
    def _sc_scatter(x_hbm, p0_hbm, p1_hbm, xs_hbm, rows_v, i0_v, i1_v,
                    sem0, sem1):
        wid = lax.axis_index("s") * 2 + lax.axis_index("c")
        base = wid * CHUNK
        pltpu.sync_copy(x_hbm.at[pl.ds(base, CHUNK)], rows_v)
        pltpu.sync_copy(p0_hbm.at[pl.ds(base, CHUNK)], i0_v)
        pltpu.sync_copy(p1_hbm.at[pl.ds(base, CHUNK)], i1_v)
        c0 = pltpu.async_copy(rows_v, xs_hbm.at[i0_v], sem0)
        c1 = pltpu.async_copy(rows_v, xs_hbm.at[i1_v], sem1)
        c0.wait()
        c1.wait()

    @functools.partial(
        pl.kernel, mesh=mesh,
        out_type=jax.ShapeDtypeStruct((N, D), jnp.float32),
        scratch_types=[
            pltpu.VMEM((CHUNK, D), jnp.float32),
            pltpu.VMEM((CHUNK, D), jnp.float32),
            pltpu.VMEM((CHUNK,), jnp.int32),
            pltpu.VMEM((CHUNK,), jnp.int32),
            pltpu.VMEM((CHUNK, 16), jnp.float32),
            pltpu.VMEM((CHUNK, 16), jnp.float32),
            pltpu.SemaphoreType.DMA,
            pltpu.SemaphoreType.DMA,
        ],
    )
    def _sc_combine(ys_hbm, p0_hbm, p1_hbm, w0_hbm, w1_hbm, out_hbm,
                    g0_v, g1_v, i0_v, i1_v, a0_v, a1_v, sem0, sem1):
        wid = lax.axis_index("s") * 2 + lax.axis_index("c")
        base = wid * CHUNK
        pltpu.sync_copy(p0_hbm.at[pl.ds(base, CHUNK)], i0_v)
        pltpu.sync_copy(p1_hbm.at[pl.ds(base, CHUNK)], i1_v)
        pltpu.sync_copy(w0_hbm.at[pl.ds(base, CHUNK)], a0_v)
        pltpu.sync_copy(w1_hbm.at[pl.ds(base, CHUNK)], a1_v)
        c0 = pltpu.async_copy(ys_hbm.at[i0_v], g0_v, sem0)
        c1 = pltpu.async_copy(ys_hbm.at[i1_v], g1_v, sem1)
        c0.wait()
        c1.wait()

        def row(i, carry):
            wa = a0_v[i, :]
            wb = a1_v[i, :]
            for j in range(D // 16):
                sl = pl.ds(j * 16, 16)
                g0_v[i, sl] = g0_v[i, sl] * wa + g1_v[i, sl] * wb
            return carry

        lax.fori_loop(0, CHUNK, row, 0)
        pltpu.sync_copy(g0_v, out_hbm.at[pl.ds(base, CHUNK)])

    return _sc_scatter, _sc_combine


def _ffn_body(bem_ref, pf_ref, sc_ref, sp_ref,
              xs_ref, w1_hbm, b1_ref, w2_hbm, b2_ref, ys_ref,
              w1s_ref, w2s_ref, w1c_ref, w2c_ref, sem1, sem2):
    b = pl.program_id(0)
    e = bem_ref[b]
    prev = jnp.where(b == 0, -2, bem_ref[jnp.maximum(b - 1, 0)])
    nxtb = jnp.where(b == NBLK - 1, -2,
                     bem_ref[jnp.minimum(b + 1, NBLK - 1)])
    fb = (e >= 0) & (e != prev)
    lb = (e >= 0) & (e != nxtb)
    pfe = pf_ref[b]

    def fetch(expert):
        pltpu.make_async_copy(w1_hbm.at[expert], w1s_ref, sem1).start()
        pltpu.make_async_copy(w2_hbm.at[expert], w2s_ref, sem2).start()

    def land(expert, slot):
        pltpu.make_async_copy(w1_hbm.at[expert], w1s_ref, sem1).wait()
        pltpu.make_async_copy(w2_hbm.at[expert], w2s_ref, sem2).wait()
        w1c_ref[slot] = w1s_ref[...].astype(jnp.bfloat16)
        w2c_ref[slot] = w2s_ref[...].astype(jnp.bfloat16)

    @pl.when(b == 0)
    def _boot():
        fetch(e)
        land(e, sc_ref[0])

    @pl.when(fb & (pfe >= 0))
    def _issue_next():
        fetch(pfe)

    @pl.when(lb & (pfe >= 0))
    def _land_next():
        land(pfe, sp_ref[b])

    @pl.when(e >= 0)
    def _compute():
        slot = sc_ref[b]
        xb = xs_ref[...].astype(jnp.bfloat16)
        h = jnp.dot(xb, w1c_ref[slot], preferred_element_type=jnp.float32)
        h = jax.nn.gelu(h + b1_ref[0])
        ys_ref[...] = jnp.dot(h.astype(jnp.bfloat16), w2c_ref[slot],
                              preferred_element_type=jnp.float32) + b2_ref[0]


def _ffn(bem, pf, sc, sp, xs, w1f, b1r, w2f, b2r):
    def bmap(b, bem_ref, pf_ref, sc_ref, sp_ref):
        e = bem_ref[b]
        return (jnp.where(e < 0, E - 1, e), 0, 0)

    grid_spec = pltpu.PrefetchScalarGridSpec(
        num_scalar_prefetch=4,
        grid=(NBLK,),
        in_specs=[
            pl.BlockSpec((BLK, D), lambda b, *_: (b, 0)),
            pl.BlockSpec(memory_space=pl.ANY),
            pl.BlockSpec((1, 1, DF), bmap),
            pl.BlockSpec(memory_space=pl.ANY),
            pl.BlockSpec((1, 1, D), bmap),
        ],
        out_specs=pl.BlockSpec((BLK, D), lambda b, *_: (b, 0)),
        scratch_shapes=[
            pltpu.VMEM((D, DF), jnp.float32),
            pltpu.VMEM((DF, D), jnp.float32),
            pltpu.VMEM((2, D, DF), jnp.bfloat16),
            pltpu.VMEM((2, DF, D), jnp.bfloat16),
            pltpu.SemaphoreType.DMA,
            pltpu.SemaphoreType.DMA,
        ],
    )
    return pl.pallas_call(
        _ffn_body,
        grid_spec=grid_spec,
        out_shape=jax.ShapeDtypeStruct((NP, D), jnp.float32),
        compiler_params=pltpu.CompilerParams(
            dimension_semantics=("arbitrary",),
        ),
    )(bem, pf, sc, sp, xs, w1f, b1r, w2f, b2r)


@jax.jit
def _run(x, router_w, w1, b1, w2, b2):
    sc_scatter, sc_combine = _sc_kernels()
    xf = x.reshape(N, D)
    pos0, pos1, w0b, w1b, bem2, pf2, sc2, sp2 = _router(xf, router_w)
    p0 = pos0.reshape(N)
    p1 = pos1.reshape(N)
    bem = bem2.reshape(NBLK)
    pf = pf2.reshape(NBLK)
    sc = sc2.reshape(NBLK)
    sp = sp2.reshape(NBLK)
    xs = sc_scatter(xf, p0, p1)
    ys = _ffn(bem, pf, sc, sp, xs, w1, b1.reshape(E, 1, DF),
              w2, b2.reshape(E, 1, D))
    out = sc_combine(ys, p0, p1, w0b, w1b)
    return out.reshape(B, T, D)


def kernel(x, router_w, w1, b1, w2, b2):
    return _run(x, router_w, w1, b1, w2, b2)


# speedup vs baseline: 1.0832x; 1.0832x over previous
import functools

import jax
import jax.numpy as jnp
from jax import lax
from jax.experimental import pallas as pl
from jax.experimental.pallas import tpu as pltpu
from jax.experimental.pallas import tpu_sc as plsc

B, T, D = 1, 2048, 768
E, K, DF = 8, 2, 3072
N = B * T
BLK = 256
NBLK = 24
NP = BLK * NBLK
CH = 128
NW = 32
CHUNK = N // NW


def _router_body(x_ref, rw_ref, pos0_ref, pos1_ref, w0_ref, w1_ref, bem_ref,
                 pf_ref, sc_ref, sp_ref):
    xf = x_ref[...]
    logits = jnp.dot(xf, rw_ref[...], preferred_element_type=jnp.float32)
    ids = jax.lax.broadcasted_iota(jnp.int32, (N, E), 1)
    m0 = jnp.max(logits, axis=1, keepdims=True)
    e0 = jnp.min(jnp.where(logits == m0, ids, E), axis=1, keepdims=True)
    oh0 = ids == e0
    l2 = jnp.where(oh0, -jnp.inf, logits)
    m1 = jnp.max(l2, axis=1, keepdims=True)
    e1 = jnp.min(jnp.where(l2 == m1, ids, E), axis=1, keepdims=True)
    oh1 = ids == e1
    t = jnp.exp(m1 - m0)
    w_hi = 1.0 / (1.0 + t)
    w_lo = 1.0 - w_hi
    ones16 = jnp.ones((1, 16), jnp.float32)
    w0_ref[...] = w_hi * ones16
    w1_ref[...] = w_lo * ones16

    f0 = oh0.astype(jnp.float32)
    f1 = oh1.astype(jnp.float32)
    sub = jax.lax.broadcasted_iota(jnp.int32, (CH, CH), 0)
    lan = jax.lax.broadcasted_iota(jnp.int32, (CH, CH), 1)
    tri = (lan < sub).astype(jnp.float32)
    run0 = jnp.zeros((1, E), jnp.float32)
    run1 = jnp.zeros((1, E), jnp.float32)
    cum0 = []
    cum1 = []
    for c in range(N // CH):
        a0 = f0[c * CH:(c + 1) * CH, :]
        a1 = f1[c * CH:(c + 1) * CH, :]
        cum0.append(jnp.dot(tri, a0, preferred_element_type=jnp.float32) + run0)
        cum1.append(jnp.dot(tri, a1, preferred_element_type=jnp.float32) + run1)
        run0 = run0 + jnp.sum(a0, axis=0, keepdims=True)
        run1 = run1 + jnp.sum(a1, axis=0, keepdims=True)
    cum0 = jnp.concatenate(cum0, axis=0)
    cum1 = jnp.concatenate(cum1, axis=0)
    cnt0 = run0
    total = run0 + run1
    padded = jnp.ceil(total / BLK) * BLK
    ii = jax.lax.broadcasted_iota(jnp.int32, (E, E), 0)
    jj = jax.lax.broadcasted_iota(jnp.int32, (E, E), 1)
    triu = (ii < jj).astype(jnp.float32)
    off = jnp.dot(padded, triu, preferred_element_type=jnp.float32)

    base0 = jnp.sum(f0 * off, axis=1, keepdims=True)
    base1 = jnp.sum(f1 * (off + cnt0), axis=1, keepdims=True)
    rank0 = jnp.sum(f0 * cum0, axis=1, keepdims=True)
    rank1 = jnp.sum(f1 * cum1, axis=1, keepdims=True)
    pos0_ref[...] = (base0 + rank0).astype(jnp.int32)
    pos1_ref[...] = (base1 + rank1).astype(jnp.int32)

    bs = jax.lax.broadcasted_iota(jnp.int32, (NBLK, 1), 0).astype(
        jnp.float32) * BLK
    bem = jnp.sum((off <= bs).astype(jnp.int32), axis=1, keepdims=True) - 1
    tot_pad = jnp.sum(padded, axis=1, keepdims=True)
    bem = jnp.where(bs < tot_pad, bem, -1)
    bem_ref[...] = bem

    present = (total > 0.0).astype(jnp.float32)
    rank = jnp.dot(present, triu, preferred_element_type=jnp.float32)
    slot_row = jnp.remainder(rank.astype(jnp.int32), 2)
    jb = jax.lax.broadcasted_iota(jnp.int32, (NBLK, E), 1)
    cand = (jb > bem) & (present > 0.0)
    pf = jnp.min(jnp.where(cand, jb, E), axis=1, keepdims=True)
    pf = jnp.where(pf == E, -1, pf)
    ohb = (jb == bem).astype(jnp.int32)
    ohp = (jb == pf).astype(jnp.int32)
    pf_ref[...] = pf
    sc_ref[...] = jnp.sum(ohb * slot_row, axis=1, keepdims=True)
    sp_ref[...] = jnp.sum(ohp * slot_row, axis=1, keepdims=True)


def _router(xf, router_w):
    return pl.pallas_call(
        _router_body,
        in_specs=[
            pl.BlockSpec((N, D), lambda: (0, 0)),
            pl.BlockSpec((D, E), lambda: (0, 0)),
        ],
        out_specs=[
            pl.BlockSpec((N, 1), lambda: (0, 0)),
            pl.BlockSpec((N, 1), lambda: (0, 0)),
            pl.BlockSpec((N, 16), lambda: (0, 0)),
            pl.BlockSpec((N, 16), lambda: (0, 0)),
            pl.BlockSpec((NBLK, 1), lambda: (0, 0)),
            pl.BlockSpec((NBLK, 1), lambda: (0, 0)),
            pl.BlockSpec((NBLK, 1), lambda: (0, 0)),
            pl.BlockSpec((NBLK, 1), lambda: (0, 0)),
        ],
        out_shape=[
            jax.ShapeDtypeStruct((N, 1), jnp.int32),
            jax.ShapeDtypeStruct((N, 1), jnp.int32),
            jax.ShapeDtypeStruct((N, 16), jnp.float32),
            jax.ShapeDtypeStruct((N, 16), jnp.float32),
            jax.ShapeDtypeStruct((NBLK, 1), jnp.int32),
            jax.ShapeDtypeStruct((NBLK, 1), jnp.int32),
            jax.ShapeDtypeStruct((NBLK, 1), jnp.int32),
            jax.ShapeDtypeStruct((NBLK, 1), jnp.int32),
        ],
    )(xf, router_w)


@functools.cache
def _sc_kernels():
    mesh = plsc.VectorSubcoreMesh(core_axis_name="c", subcore_axis_name="s")

    @functools.partial(
        pl.kernel, mesh=mesh,
        out_type=jax.ShapeDtypeStruct((NP, D), jnp.float32),
        scratch_types=[
            pltpu.VMEM((CHUNK, D), jnp.float32),
            pltpu.VMEM((CHUNK,), jnp.int32),
            pltpu.VMEM((CHUNK,), jnp.int32),
            pltpu.SemaphoreType.DMA,
            pltpu.SemaphoreType.DMA,
        ],
    )
    def _sc_scatter(x_hbm, p0_hbm, p1_hbm, xs_hbm, rows_v, i0_v, i1_v,
                    sem0, sem1):
        wid = lax.axis_index("s") * 2 + lax.axis_index("c")
        base = wid * CHUNK
        pltpu.sync_copy(x_hbm.at[pl.ds(base, CHUNK)], rows_v)
        pltpu.sync_copy(p0_hbm.at[pl.ds(base, CHUNK)], i0_v)
        pltpu.sync_copy(p1_hbm.at[pl.ds(base, CHUNK)], i1_v)
        c0 = pltpu.async_copy(rows_v, xs_hbm.at[i0_v], sem0)
        c1 = pltpu.async_copy(rows_v, xs_hbm.at[i1_v], sem1)
        c0.wait()
        c1.wait()

    @functools.partial(
        pl.kernel, mesh=mesh,
        out_type=jax.ShapeDtypeStruct((N, D), jnp.float32),
        scratch_types=[
            pltpu.VMEM((CHUNK, D), jnp.float32),
            pltpu.VMEM((CHUNK, D), jnp.float32),
            pltpu.VMEM((CHUNK,), jnp.int32),
            pltpu.VMEM((CHUNK,), jnp.int32),
            pltpu.VMEM((CHUNK, 16), jnp.float32),
            pltpu.VMEM((CHUNK, 16), jnp.float32),
            pltpu.SemaphoreType.DMA,
            pltpu.SemaphoreType.DMA,
        ],
    )
    def _sc_combine(ys_hbm, p0_hbm, p1_hbm, w0_hbm, w1_hbm, out_hbm,
                    g0_v, g1_v, i0_v, i1_v, a0_v, a1_v, sem0, sem1):
        wid = lax.axis_index("s") * 2 + lax.axis_index("c")
        base = wid * CHUNK
        pltpu.sync_copy(p0_hbm.at[pl.ds(base, CHUNK)], i0_v)
        pltpu.sync_copy(p1_hbm.at[pl.ds(base, CHUNK)], i1_v)
        pltpu.sync_copy(w0_hbm.at[pl.ds(base, CHUNK)], a0_v)
        pltpu.sync_copy(w1_hbm.at[pl.ds(base, CHUNK)], a1_v)
        c0 = pltpu.async_copy(ys_hbm.at[i0_v], g0_v, sem0)
        c1 = pltpu.async_copy(ys_hbm.at[i1_v], g1_v, sem1)
        c0.wait()
        c1.wait()

        def row(i, carry):
            wa = a0_v[i, :]
            wb = a1_v[i, :]
            for j in range(D // 16):
                sl = pl.ds(j * 16, 16)
                g0_v[i, sl] = g0_v[i, sl] * wa + g1_v[i, sl] * wb
            return carry

        lax.fori_loop(0, CHUNK, row, 0)
        pltpu.sync_copy(g0_v, out_hbm.at[pl.ds(base, CHUNK)])

    return _sc_scatter, _sc_combine


def _ffn_body(bem_ref, pf_ref, sc_ref, sp_ref,
              xs_ref, w1_hbm, b1_ref, w2_hbm, b2_ref, ys_ref,
              w1s_ref, w2s_ref, w1c_ref, w2c_ref, sem1, sem2):
    b = pl.program_id(0)
    e = bem_ref[b]
    prev = jnp.where(b == 0, -2, bem_ref[jnp.maximum(b - 1, 0)])
    nxtb = jnp.where(b == NBLK - 1, -2,
                     bem_ref[jnp.minimum(b + 1, NBLK - 1)])
    fb = (e >= 0) & (e != prev)
    lb = (e >= 0) & (e != nxtb)
    pfe = pf_ref[b]

    def fetch(expert):
        pltpu.make_async_copy(w1_hbm.at[expert], w1s_ref, sem1).start()
        pltpu.make_async_copy(w2_hbm.at[expert], w2s_ref, sem2).start()

    def land(expert, slot):
        pltpu.make_async_copy(w1_hbm.at[expert], w1s_ref, sem1).wait()
        pltpu.make_async_copy(w2_hbm.at[expert], w2s_ref, sem2).wait()
        w1c_ref[slot] = w1s_ref[...].astype(jnp.bfloat16)
        w2c_ref[slot] = w2s_ref[...].astype(jnp.bfloat16)

    @pl.when(b == 0)
    def _boot():
        fetch(e)
        land(e, sc_ref[0])

    @pl.when(fb & (pfe >= 0))
    def _issue_next():
        fetch(pfe)

    @pl.when(e >= 0)
    def _compute():
        slot = sc_ref[b]
        xb = xs_ref[...].astype(jnp.bfloat16)
        h = jnp.dot(xb, w1c_ref[slot], preferred_element_type=jnp.float32)
        h = jax.nn.gelu(h + b1_ref[0])
        ys_ref[...] = jnp.dot(h.astype(jnp.bfloat16), w2c_ref[slot],
                              preferred_element_type=jnp.float32) + b2_ref[0]

    @pl.when(lb & (pfe >= 0))
    def _land_next():
        land(pfe, sp_ref[b])


def _ffn(bem, pf, sc, sp, xs, w1f, b1r, w2f, b2r):
    def bmap(b, bem_ref, pf_ref, sc_ref, sp_ref):
        e = bem_ref[b]
        return (jnp.where(e < 0, E - 1, e), 0, 0)

    grid_spec = pltpu.PrefetchScalarGridSpec(
        num_scalar_prefetch=4,
        grid=(NBLK,),
        in_specs=[
            pl.BlockSpec((BLK, D), lambda b, *_: (b, 0)),
            pl.BlockSpec(memory_space=pl.ANY),
            pl.BlockSpec((1, 1, DF), bmap),
            pl.BlockSpec(memory_space=pl.ANY),
            pl.BlockSpec((1, 1, D), bmap),
        ],
        out_specs=pl.BlockSpec((BLK, D), lambda b, *_: (b, 0)),
        scratch_shapes=[
            pltpu.VMEM((D, DF), jnp.float32),
            pltpu.VMEM((DF, D), jnp.float32),
            pltpu.VMEM((2, D, DF), jnp.bfloat16),
            pltpu.VMEM((2, DF, D), jnp.bfloat16),
            pltpu.SemaphoreType.DMA,
            pltpu.SemaphoreType.DMA,
        ],
    )
    return pl.pallas_call(
        _ffn_body,
        grid_spec=grid_spec,
        out_shape=jax.ShapeDtypeStruct((NP, D), jnp.float32),
        compiler_params=pltpu.CompilerParams(
            dimension_semantics=("arbitrary",),
        ),
    )(bem, pf, sc, sp, xs, w1f, b1r, w2f, b2r)


@jax.jit
def _run(x, router_w, w1, b1, w2, b2):
    sc_scatter, sc_combine = _sc_kernels()
    xf = x.reshape(N, D)
    pos0, pos1, w0b, w1b, bem2, pf2, sc2, sp2 = _router(xf, router_w)
    p0 = pos0.reshape(N)
    p1 = pos1.reshape(N)
    bem = bem2.reshape(NBLK)
    pf = pf2.reshape(NBLK)
    sc = sc2.reshape(NBLK)
    sp = sp2.reshape(NBLK)
    xs = sc_scatter(xf, p0, p1)
    ys = _ffn(bem, pf, sc, sp, xs, w1, b1.reshape(E, 1, DF),
              w2, b2.reshape(E, 1, D))
    out = sc_combine(ys, p0, p1, w0b, w1b)
    return out.reshape(B, T, D)


def kernel(x, router_w, w1, b1, w2, b2):
    return _run(x, router_w, w1, b1, w2, b2)


# speedup vs baseline: 1.1311x; 1.0442x over previous
import functools

import jax
import jax.numpy as jnp
from jax import lax
from jax.experimental import pallas as pl
from jax.experimental.pallas import tpu as pltpu
from jax.experimental.pallas import tpu_sc as plsc

B, T, D = 1, 2048, 768
E, K, DF = 8, 2, 3072
N = B * T
BLK = 256
NBLK = 24
NP = BLK * NBLK
CH = 128
NW = 32
CHUNK = N // NW


def _router_body(x_ref, rw_ref, pos0_ref, pos1_ref, w0_ref, w1_ref, bem_ref,
                 pf_ref, sc_ref, sp_ref):
    xf = x_ref[...]
    logits = jnp.dot(xf, rw_ref[...], preferred_element_type=jnp.float32)
    ids = jax.lax.broadcasted_iota(jnp.int32, (N, E), 1)
    m0 = jnp.max(logits, axis=1, keepdims=True)
    e0 = jnp.min(jnp.where(logits == m0, ids, E), axis=1, keepdims=True)
    oh0 = ids == e0
    l2 = jnp.where(oh0, -jnp.inf, logits)
    m1 = jnp.max(l2, axis=1, keepdims=True)
    e1 = jnp.min(jnp.where(l2 == m1, ids, E), axis=1, keepdims=True)
    oh1 = ids == e1
    t = jnp.exp(m1 - m0)
    w_hi = 1.0 / (1.0 + t)
    w_lo = 1.0 - w_hi
    ones16 = jnp.ones((1, 16), jnp.float32)
    w0_ref[...] = w_hi * ones16
    w1_ref[...] = w_lo * ones16

    f0 = oh0.astype(jnp.float32)
    f1 = oh1.astype(jnp.float32)
    sub = jax.lax.broadcasted_iota(jnp.int32, (CH, CH), 0)
    lan = jax.lax.broadcasted_iota(jnp.int32, (CH, CH), 1)
    tri = (lan < sub).astype(jnp.float32)
    run0 = jnp.zeros((1, E), jnp.float32)
    run1 = jnp.zeros((1, E), jnp.float32)
    cum0 = []
    cum1 = []
    for c in range(N // CH):
        a0 = f0[c * CH:(c + 1) * CH, :]
        a1 = f1[c * CH:(c + 1) * CH, :]
        cum0.append(jnp.dot(tri, a0, preferred_element_type=jnp.float32) + run0)
        cum1.append(jnp.dot(tri, a1, preferred_element_type=jnp.float32) + run1)
        run0 = run0 + jnp.sum(a0, axis=0, keepdims=True)
        run1 = run1 + jnp.sum(a1, axis=0, keepdims=True)
    cum0 = jnp.concatenate(cum0, axis=0)
    cum1 = jnp.concatenate(cum1, axis=0)
    cnt0 = run0
    total = run0 + run1
    padded = jnp.ceil(total / BLK) * BLK
    ii = jax.lax.broadcasted_iota(jnp.int32, (E, E), 0)
    jj = jax.lax.broadcasted_iota(jnp.int32, (E, E), 1)
    triu = (ii < jj).astype(jnp.float32)
    off = jnp.dot(padded, triu, preferred_element_type=jnp.float32)

    base0 = jnp.sum(f0 * off, axis=1, keepdims=True)
    base1 = jnp.sum(f1 * (off + cnt0), axis=1, keepdims=True)
    rank0 = jnp.sum(f0 * cum0, axis=1, keepdims=True)
    rank1 = jnp.sum(f1 * cum1, axis=1, keepdims=True)
    pos0_ref[...] = (base0 + rank0).astype(jnp.int32)
    pos1_ref[...] = (base1 + rank1).astype(jnp.int32)

    bs = jax.lax.broadcasted_iota(jnp.int32, (NBLK, 1), 0).astype(
        jnp.float32) * BLK
    bem = jnp.sum((off <= bs).astype(jnp.int32), axis=1, keepdims=True) - 1
    tot_pad = jnp.sum(padded, axis=1, keepdims=True)
    bem = jnp.where(bs < tot_pad, bem, -1)
    bem_ref[...] = bem

    present = (total > 0.0).astype(jnp.float32)
    rank = jnp.dot(present, triu, preferred_element_type=jnp.float32)
    slot_row = jnp.remainder(rank.astype(jnp.int32), 2)
    jb = jax.lax.broadcasted_iota(jnp.int32, (NBLK, E), 1)
    cand = (jb > bem) & (present > 0.0)
    pf = jnp.min(jnp.where(cand, jb, E), axis=1, keepdims=True)
    pf = jnp.where(pf == E, -1, pf)
    ohb = (jb == bem).astype(jnp.int32)
    ohp = (jb == pf).astype(jnp.int32)
    pf_ref[...] = pf
    sc_ref[...] = jnp.sum(ohb * slot_row, axis=1, keepdims=True)
    sp_ref[...] = jnp.sum(ohp * slot_row, axis=1, keepdims=True)


def _router(xf, router_w):
    return pl.pallas_call(
        _router_body,
        in_specs=[
            pl.BlockSpec((N, D), lambda: (0, 0)),
            pl.BlockSpec((D, E), lambda: (0, 0)),
        ],
        out_specs=[
            pl.BlockSpec((N, 1), lambda: (0, 0)),
            pl.BlockSpec((N, 1), lambda: (0, 0)),
            pl.BlockSpec((N, 16), lambda: (0, 0)),
            pl.BlockSpec((N, 16), lambda: (0, 0)),
            pl.BlockSpec((NBLK, 1), lambda: (0, 0)),
            pl.BlockSpec((NBLK, 1), lambda: (0, 0)),
            pl.BlockSpec((NBLK, 1), lambda: (0, 0)),
            pl.BlockSpec((NBLK, 1), lambda: (0, 0)),
        ],
        out_shape=[
            jax.ShapeDtypeStruct((N, 1), jnp.int32),
            jax.ShapeDtypeStruct((N, 1), jnp.int32),
            jax.ShapeDtypeStruct((N, 16), jnp.float32),
            jax.ShapeDtypeStruct((N, 16), jnp.float32),
            jax.ShapeDtypeStruct((NBLK, 1), jnp.int32),
            jax.ShapeDtypeStruct((NBLK, 1), jnp.int32),
            jax.ShapeDtypeStruct((NBLK, 1), jnp.int32),
            jax.ShapeDtypeStruct((NBLK, 1), jnp.int32),
        ],
    )(xf, router_w)


@functools.cache
def _sc_kernels():
    mesh = plsc.VectorSubcoreMesh(core_axis_name="c", subcore_axis_name="s")

    @functools.partial(
        pl.kernel, mesh=mesh,
        out_type=jax.ShapeDtypeStruct((NP, D), jnp.float32),
        scratch_types=[
            pltpu.VMEM((CHUNK, D), jnp.float32),
            pltpu.VMEM((CHUNK,), jnp.int32),
            pltpu.VMEM((CHUNK,), jnp.int32),
            pltpu.SemaphoreType.DMA,
            pltpu.SemaphoreType.DMA,
        ],
    )
    def _sc_scatter(x_hbm, p0_hbm, p1_hbm, xs_hbm, rows_v, i0_v, i1_v,
                    sem0, sem1):
        wid = lax.axis_index("s") * 2 + lax.axis_index("c")
        base = wid * CHUNK
        pltpu.sync_copy(x_hbm.at[pl.ds(base, CHUNK)], rows_v)
        pltpu.sync_copy(p0_hbm.at[pl.ds(base, CHUNK)], i0_v)
        pltpu.sync_copy(p1_hbm.at[pl.ds(base, CHUNK)], i1_v)
        c0 = pltpu.async_copy(rows_v, xs_hbm.at[i0_v], sem0)
        c1 = pltpu.async_copy(rows_v, xs_hbm.at[i1_v], sem1)
        c0.wait()
        c1.wait()

    @functools.partial(
        pl.kernel, mesh=mesh,
        out_type=jax.ShapeDtypeStruct((N, D), jnp.float32),
        scratch_types=[
            pltpu.VMEM((CHUNK, D), jnp.float32),
            pltpu.VMEM((CHUNK, D), jnp.float32),
            pltpu.VMEM((CHUNK,), jnp.int32),
            pltpu.VMEM((CHUNK,), jnp.int32),
            pltpu.VMEM((CHUNK, 16), jnp.float32),
            pltpu.VMEM((CHUNK, 16), jnp.float32),
            pltpu.SemaphoreType.DMA,
            pltpu.SemaphoreType.DMA,
        ],
    )
    def _sc_combine(ys_hbm, p0_hbm, p1_hbm, w0_hbm, w1_hbm, out_hbm,
                    g0_v, g1_v, i0_v, i1_v, a0_v, a1_v, sem0, sem1):
        wid = lax.axis_index("s") * 2 + lax.axis_index("c")
        base = wid * CHUNK
        pltpu.sync_copy(p0_hbm.at[pl.ds(base, CHUNK)], i0_v)
        pltpu.sync_copy(p1_hbm.at[pl.ds(base, CHUNK)], i1_v)
        pltpu.sync_copy(w0_hbm.at[pl.ds(base, CHUNK)], a0_v)
        pltpu.sync_copy(w1_hbm.at[pl.ds(base, CHUNK)], a1_v)
        c0 = pltpu.async_copy(ys_hbm.at[i0_v], g0_v, sem0)
        c1 = pltpu.async_copy(ys_hbm.at[i1_v], g1_v, sem1)
        c0.wait()
        c1.wait()

        def row(i, carry):
            wa = a0_v[i, :]
            wb = a1_v[i, :]
            for j in range(D // 16):
                sl = pl.ds(j * 16, 16)
                g0_v[i, sl] = g0_v[i, sl] * wa + g1_v[i, sl] * wb
            return carry

        lax.fori_loop(0, CHUNK, row, 0)
        pltpu.sync_copy(g0_v, out_hbm.at[pl.ds(base, CHUNK)])

    return _sc_scatter, _sc_combine


def _ffn_body(bem_ref, pf_ref, sc_ref, sp_ref,
              xs_ref, w1_hbm, b1_ref, w2_hbm, b2_ref, ys_ref,
              w1s_ref, w2s_ref, w1c_ref, w2c_ref, sem1, sem2):
    b = pl.program_id(0)
    e = bem_ref[b]
    prev = jnp.where(b == 0, -2, bem_ref[jnp.maximum(b - 1, 0)])
    fb = (e >= 0) & (e != prev)
    pfe = pf_ref[b]

    def start(expert, slot):
        pltpu.make_async_copy(w1_hbm.at[expert], w1s_ref.at[slot],
                              sem1.at[slot]).start()
        pltpu.make_async_copy(w2_hbm.at[expert], w2s_ref.at[slot],
                              sem2.at[slot]).start()

    @pl.when(b == 0)
    def _fetch_first():
        start(e, sc_ref[0])

    @pl.when(fb & (pfe >= 0))
    def _prefetch_next():
        start(pfe, sp_ref[b])

    @pl.when(fb)
    def _wait_and_cast():
        slot = sc_ref[b]
        pltpu.make_async_copy(w1_hbm.at[e], w1s_ref.at[slot],
                              sem1.at[slot]).wait()
        pltpu.make_async_copy(w2_hbm.at[e], w2s_ref.at[slot],
                              sem2.at[slot]).wait()
        w1c_ref[...] = w1s_ref[slot].astype(jnp.bfloat16)
        w2c_ref[...] = w2s_ref[slot].astype(jnp.bfloat16)

    @pl.when(e >= 0)
    def _compute():
        xb = xs_ref[...].astype(jnp.bfloat16)
        h = jnp.dot(xb, w1c_ref[...], preferred_element_type=jnp.float32)
        h = jax.nn.gelu(h + b1_ref[0])
        ys_ref[...] = jnp.dot(h.astype(jnp.bfloat16), w2c_ref[...],
                              preferred_element_type=jnp.float32) + b2_ref[0]


def _ffn(bem, pf, sc, sp, xs, w1f, b1r, w2f, b2r):
    def bmap(b, bem_ref, pf_ref, sc_ref, sp_ref):
        e = bem_ref[b]
        return (jnp.where(e < 0, E - 1, e), 0, 0)

    grid_spec = pltpu.PrefetchScalarGridSpec(
        num_scalar_prefetch=4,
        grid=(NBLK,),
        in_specs=[
            pl.BlockSpec((BLK, D), lambda b, *_: (b, 0)),
            pl.BlockSpec(memory_space=pl.ANY),
            pl.BlockSpec((1, 1, DF), bmap),
            pl.BlockSpec(memory_space=pl.ANY),
            pl.BlockSpec((1, 1, D), bmap),
        ],
        out_specs=pl.BlockSpec((BLK, D), lambda b, *_: (b, 0)),
        scratch_shapes=[
            pltpu.VMEM((2, D, DF), jnp.float32),
            pltpu.VMEM((2, DF, D), jnp.float32),
            pltpu.VMEM((D, DF), jnp.bfloat16),
            pltpu.VMEM((DF, D), jnp.bfloat16),
            pltpu.SemaphoreType.DMA((2,)),
            pltpu.SemaphoreType.DMA((2,)),
        ],
    )
    return pl.pallas_call(
        _ffn_body,
        grid_spec=grid_spec,
        out_shape=jax.ShapeDtypeStruct((NP, D), jnp.float32),
        compiler_params=pltpu.CompilerParams(
            dimension_semantics=("arbitrary",),
        ),
    )(bem, pf, sc, sp, xs, w1f, b1r, w2f, b2r)


@jax.jit
def _run(x, router_w, w1, b1, w2, b2):
    sc_scatter, sc_combine = _sc_kernels()
    xf = x.reshape(N, D)
    pos0, pos1, w0b, w1b, bem2, pf2, sc2, sp2 = _router(xf, router_w)
    p0 = pos0.reshape(N)
    p1 = pos1.reshape(N)
    bem = bem2.reshape(NBLK)
    pf = pf2.reshape(NBLK)
    sc = sc2.reshape(NBLK)
    sp = sp2.reshape(NBLK)
    xs = sc_scatter(xf, p0, p1)
    ys = _ffn(bem, pf, sc, sp, xs, w1, b1.reshape(E, 1, DF),
              w2, b2.reshape(E, 1, D))
    out = sc_combine(ys, p0, p1, w0b, w1b)
    return out.reshape(B, T, D)


def kernel(x, router_w, w1, b1, w2, b2):
    return _run(x, router_w, w1, b1, w2, b2)


# speedup vs baseline: 1.1416x; 1.0093x over previous
import functools

import jax
import jax.numpy as jnp
from jax import lax
from jax.experimental import pallas as pl
from jax.experimental.pallas import tpu as pltpu
from jax.experimental.pallas import tpu_sc as plsc

B, T, D = 1, 2048, 768
E, K, DF = 8, 2, 3072
N = B * T
BLK = 256
NBLK = 23
NP = BLK * NBLK
CH = 128
NW = 32
CHUNK = N // NW


def _router_body(x_ref, rw_ref, pos0_ref, pos1_ref, w0_ref, w1_ref, bem_ref,
                 pf_ref, sc_ref, sp_ref):
    xf = x_ref[0]
    logits = jnp.dot(xf, rw_ref[...], preferred_element_type=jnp.float32)
    ids = jax.lax.broadcasted_iota(jnp.int32, (N, E), 1)
    m0 = jnp.max(logits, axis=1, keepdims=True)
    e0 = jnp.min(jnp.where(logits == m0, ids, E), axis=1, keepdims=True)
    oh0 = ids == e0
    l2 = jnp.where(oh0, -jnp.inf, logits)
    m1 = jnp.max(l2, axis=1, keepdims=True)
    e1 = jnp.min(jnp.where(l2 == m1, ids, E), axis=1, keepdims=True)
    oh1 = ids == e1
    t = jnp.exp(m1 - m0)
    w_hi = 1.0 / (1.0 + t)
    w_lo = 1.0 - w_hi
    ones16 = jnp.ones((1, 16), jnp.float32)
    w0_ref[...] = w_hi * ones16
    w1_ref[...] = w_lo * ones16

    f0 = oh0.astype(jnp.float32)
    f1 = oh1.astype(jnp.float32)
    sub = jax.lax.broadcasted_iota(jnp.int32, (CH, CH), 0)
    lan = jax.lax.broadcasted_iota(jnp.int32, (CH, CH), 1)
    tri = (lan < sub).astype(jnp.float32)
    run0 = jnp.zeros((1, E), jnp.float32)
    run1 = jnp.zeros((1, E), jnp.float32)
    cum0 = []
    cum1 = []
    for c in range(N // CH):
        a0 = f0[c * CH:(c + 1) * CH, :]
        a1 = f1[c * CH:(c + 1) * CH, :]
        cum0.append(jnp.dot(tri, a0, preferred_element_type=jnp.float32) + run0)
        cum1.append(jnp.dot(tri, a1, preferred_element_type=jnp.float32) + run1)
        run0 = run0 + jnp.sum(a0, axis=0, keepdims=True)
        run1 = run1 + jnp.sum(a1, axis=0, keepdims=True)
    cum0 = jnp.concatenate(cum0, axis=0)
    cum1 = jnp.concatenate(cum1, axis=0)
    cnt0 = run0
    total = run0 + run1
    padded = jnp.ceil(total / BLK) * BLK
    ii = jax.lax.broadcasted_iota(jnp.int32, (E, E), 0)
    jj = jax.lax.broadcasted_iota(jnp.int32, (E, E), 1)
    triu = (ii < jj).astype(jnp.float32)
    off = jnp.dot(padded, triu, preferred_element_type=jnp.float32)

    base0 = jnp.sum(f0 * off, axis=1, keepdims=True)
    base1 = jnp.sum(f1 * (off + cnt0), axis=1, keepdims=True)
    rank0 = jnp.sum(f0 * cum0, axis=1, keepdims=True)
    rank1 = jnp.sum(f1 * cum1, axis=1, keepdims=True)
    pos0_ref[...] = (base0 + rank0).astype(jnp.int32)
    pos1_ref[...] = (base1 + rank1).astype(jnp.int32)

    bs = jax.lax.broadcasted_iota(jnp.int32, (NBLK, 1), 0).astype(
        jnp.float32) * BLK
    bem = jnp.sum((off <= bs).astype(jnp.int32), axis=1, keepdims=True) - 1
    tot_pad = jnp.sum(padded, axis=1, keepdims=True)
    bem = jnp.where(bs < tot_pad, bem, -1)
    bem_ref[...] = bem

    present = (total > 0.0).astype(jnp.float32)
    rank = jnp.dot(present, triu, preferred_element_type=jnp.float32)
    slot_row = jnp.remainder(rank.astype(jnp.int32), 2)
    jb = jax.lax.broadcasted_iota(jnp.int32, (NBLK, E), 1)
    cand = (jb > bem) & (present > 0.0)
    pf = jnp.min(jnp.where(cand, jb, E), axis=1, keepdims=True)
    pf = jnp.where(pf == E, -1, pf)
    ohb = (jb == bem).astype(jnp.int32)
    ohp = (jb == pf).astype(jnp.int32)
    pf_ref[...] = pf
    sc_ref[...] = jnp.sum(ohb * slot_row, axis=1, keepdims=True)
    sp_ref[...] = jnp.sum(ohp * slot_row, axis=1, keepdims=True)


def _router(x, router_w):
    return pl.pallas_call(
        _router_body,
        in_specs=[
            pl.BlockSpec((1, N, D), lambda: (0, 0, 0)),
            pl.BlockSpec((D, E), lambda: (0, 0)),
        ],
        out_specs=[
            pl.BlockSpec((N, 1), lambda: (0, 0)),
            pl.BlockSpec((N, 1), lambda: (0, 0)),
            pl.BlockSpec((N, 16), lambda: (0, 0)),
            pl.BlockSpec((N, 16), lambda: (0, 0)),
            pl.BlockSpec((NBLK, 1), lambda: (0, 0)),
            pl.BlockSpec((NBLK, 1), lambda: (0, 0)),
            pl.BlockSpec((NBLK, 1), lambda: (0, 0)),
            pl.BlockSpec((NBLK, 1), lambda: (0, 0)),
        ],
        out_shape=[
            jax.ShapeDtypeStruct((N, 1), jnp.int32),
            jax.ShapeDtypeStruct((N, 1), jnp.int32),
            jax.ShapeDtypeStruct((N, 16), jnp.float32),
            jax.ShapeDtypeStruct((N, 16), jnp.float32),
            jax.ShapeDtypeStruct((NBLK, 1), jnp.int32),
            jax.ShapeDtypeStruct((NBLK, 1), jnp.int32),
            jax.ShapeDtypeStruct((NBLK, 1), jnp.int32),
            jax.ShapeDtypeStruct((NBLK, 1), jnp.int32),
        ],
    )(x, router_w)


@functools.cache
def _sc_kernels():
    mesh = plsc.VectorSubcoreMesh(core_axis_name="c", subcore_axis_name="s")

    @functools.partial(
        pl.kernel, mesh=mesh,
        out_type=jax.ShapeDtypeStruct((NP, D), jnp.float32),
        scratch_types=[
            pltpu.VMEM((CHUNK, D), jnp.float32),
            pltpu.VMEM((CHUNK,), jnp.int32),
            pltpu.VMEM((CHUNK,), jnp.int32),
            pltpu.SemaphoreType.DMA,
            pltpu.SemaphoreType.DMA,
        ],
    )
    def _sc_scatter(x_hbm, p0_hbm, p1_hbm, xs_hbm, rows_v, i0_v, i1_v,
                    sem0, sem1):
        wid = lax.axis_index("s") * 2 + lax.axis_index("c")
        base = wid * CHUNK
        pltpu.sync_copy(x_hbm.at[0, pl.ds(base, CHUNK)], rows_v)
        pltpu.sync_copy(p0_hbm.at[pl.ds(base, CHUNK)], i0_v)
        pltpu.sync_copy(p1_hbm.at[pl.ds(base, CHUNK)], i1_v)
        c0 = pltpu.async_copy(rows_v, xs_hbm.at[i0_v], sem0)
        c1 = pltpu.async_copy(rows_v, xs_hbm.at[i1_v], sem1)
        c0.wait()
        c1.wait()

    @functools.partial(
        pl.kernel, mesh=mesh,
        out_type=jax.ShapeDtypeStruct((N, D), jnp.float32),
        scratch_types=[
            pltpu.VMEM((CHUNK, D), jnp.float32),
            pltpu.VMEM((CHUNK, D), jnp.float32),
            pltpu.VMEM((CHUNK,), jnp.int32),
            pltpu.VMEM((CHUNK,), jnp.int32),
            pltpu.VMEM((CHUNK, 16), jnp.float32),
            pltpu.VMEM((CHUNK, 16), jnp.float32),
            pltpu.SemaphoreType.DMA,
            pltpu.SemaphoreType.DMA,
        ],
    )
    def _sc_combine(ys_hbm, p0_hbm, p1_hbm, w0_hbm, w1_hbm, out_hbm,
                    g0_v, g1_v, i0_v, i1_v, a0_v, a1_v, sem0, sem1):
        wid = lax.axis_index("s") * 2 + lax.axis_index("c")
        base = wid * CHUNK
        pltpu.sync_copy(p0_hbm.at[pl.ds(base, CHUNK)], i0_v)
        pltpu.sync_copy(p1_hbm.at[pl.ds(base, CHUNK)], i1_v)
        pltpu.sync_copy(w0_hbm.at[pl.ds(base, CHUNK)], a0_v)
        pltpu.sync_copy(w1_hbm.at[pl.ds(base, CHUNK)], a1_v)
        c0 = pltpu.async_copy(ys_hbm.at[i0_v], g0_v, sem0)
        c1 = pltpu.async_copy(ys_hbm.at[i1_v], g1_v, sem1)
        c0.wait()
        c1.wait()

        def row(i, carry):
            wa = a0_v[i, :]
            wb = a1_v[i, :]
            for j in range(D // 16):
                sl = pl.ds(j * 16, 16)
                g0_v[i, sl] = g0_v[i, sl] * wa + g1_v[i, sl] * wb
            return carry

        lax.fori_loop(0, CHUNK, row, 0)
        pltpu.sync_copy(g0_v, out_hbm.at[pl.ds(base, CHUNK)])

    return _sc_scatter, _sc_combine


def _ffn_body(bem_ref, pf_ref, sc_ref, sp_ref,
              xs_ref, w1_hbm, b1_ref, w2_hbm, b2_ref, ys_ref,
              w1s_ref, w2s_ref, w1c_ref, w2c_ref, sem1, sem2):
    b = pl.program_id(0)
    e = bem_ref[b]
    prev = jnp.where(b == 0, -2, bem_ref[jnp.maximum(b - 1, 0)])
    fb = (e >= 0) & (e != prev)
    pfe = pf_ref[b]

    def start(expert, slot):
        pltpu.make_async_copy(w1_hbm.at[expert], w1s_ref.at[slot],
                              sem1.at[slot]).start()
        pltpu.make_async_copy(w2_hbm.at[expert], w2s_ref.at[slot],
                              sem2.at[slot]).start()

    @pl.when(b == 0)
    def _fetch_first():
        start(e, sc_ref[0])

    @pl.when(fb & (pfe >= 0))
    def _prefetch_next():
        start(pfe, sp_ref[b])

    @pl.when(fb)
    def _wait_and_cast():
        slot = sc_ref[b]
        pltpu.make_async_copy(w1_hbm.at[e], w1s_ref.at[slot],
                              sem1.at[slot]).wait()
        pltpu.make_async_copy(w2_hbm.at[e], w2s_ref.at[slot],
                              sem2.at[slot]).wait()
        w1c_ref[...] = w1s_ref[slot].astype(jnp.bfloat16)
        w2c_ref[...] = w2s_ref[slot].astype(jnp.bfloat16)

    @pl.when(e >= 0)
    def _compute():
        xb = xs_ref[...].astype(jnp.bfloat16)
        h = jnp.dot(xb, w1c_ref[...], preferred_element_type=jnp.float32)
        h = jax.nn.gelu(h + b1_ref[0])
        ys_ref[...] = jnp.dot(h.astype(jnp.bfloat16), w2c_ref[...],
                              preferred_element_type=jnp.float32) + b2_ref[0]


def _ffn(bem, pf, sc, sp, xs, w1f, b1r, w2f, b2r):
    def bmap(b, bem_ref, pf_ref, sc_ref, sp_ref):
        e = bem_ref[b]
        return (jnp.where(e < 0, E - 1, e), 0, 0)

    grid_spec = pltpu.PrefetchScalarGridSpec(
        num_scalar_prefetch=4,
        grid=(NBLK,),
        in_specs=[
            pl.BlockSpec((BLK, D), lambda b, *_: (b, 0)),
            pl.BlockSpec(memory_space=pl.ANY),
            pl.BlockSpec((1, 1, DF), bmap),
            pl.BlockSpec(memory_space=pl.ANY),
            pl.BlockSpec((1, 1, D), bmap),
        ],
        out_specs=pl.BlockSpec((BLK, D), lambda b, *_: (b, 0)),
        scratch_shapes=[
            pltpu.VMEM((2, D, DF), jnp.float32),
            pltpu.VMEM((2, DF, D), jnp.float32),
            pltpu.VMEM((D, DF), jnp.bfloat16),
            pltpu.VMEM((DF, D), jnp.bfloat16),
            pltpu.SemaphoreType.DMA((2,)),
            pltpu.SemaphoreType.DMA((2,)),
        ],
    )
    return pl.pallas_call(
        _ffn_body,
        grid_spec=grid_spec,
        out_shape=jax.ShapeDtypeStruct((NP, D), jnp.float32),
        compiler_params=pltpu.CompilerParams(
            dimension_semantics=("arbitrary",),
        ),
    )(bem, pf, sc, sp, xs, w1f, b1r, w2f, b2r)


@jax.jit
def _run(x, router_w, w1, b1, w2, b2):
    sc_scatter, sc_combine = _sc_kernels()
    pos0, pos1, w0b, w1b, bem2, pf2, sc2, sp2 = _router(x, router_w)
    p0 = pos0.reshape(N)
    p1 = pos1.reshape(N)
    bem = bem2.reshape(NBLK)
    pf = pf2.reshape(NBLK)
    sc = sc2.reshape(NBLK)
    sp = sp2.reshape(NBLK)
    xs = sc_scatter(x, p0, p1)
    ys = _ffn(bem, pf, sc, sp, xs, w1, b1.reshape(E, 1, DF),
              w2, b2.reshape(E, 1, D))
    out = sc_combine(ys, p0, p1, w0b, w1b)
    return out.reshape(B, T, D)


def kernel(x, router_w, w1, b1, w2, b2):
    return _run(x, router_w, w1, b1, w2, b2)
